# baseline (speedup 1.0000x reference)
import functools

import jax
import jax.numpy as jnp
from jax import lax
from jax.experimental import pallas as pl
from jax.experimental.pallas import tpu as pltpu
from jax.experimental.pallas import tpu_sc as plsc

CHUNK = 128
NBUF = 7
KG = 5


@functools.cache
def _build(n_rows, d):
    info = plsc.get_sparse_core_info()
    nc, ns = info.num_cores, info.num_subcores
    nw = nc * ns
    rows_per_w = n_rows // nw
    n_ch = rows_per_w // CHUNK
    assert rows_per_w * nw == n_rows and n_ch * CHUNK == rows_per_w
    assert n_ch > 2 * NBUF

    mesh = plsc.VectorSubcoreMesh(core_axis_name="c", subcore_axis_name="s")

    @functools.partial(
        pl.kernel,
        out_type=jax.ShapeDtypeStruct((n_rows, d), jnp.float32),
        mesh=mesh,
        scratch_types=(
            [pltpu.VMEM((NBUF, CHUNK), jnp.int32),
             pltpu.VMEM((NBUF, CHUNK, d), jnp.float32)]
            + [pltpu.SemaphoreType.DMA] * (3 * NBUF)
        ),
    )
    def gather(idx_hbm, table_hbm, out_hbm, idx_v, rows_v, *sems):
        isem = sems[0:NBUF]
        gsem = sems[NBUF:2 * NBUF]
        wsem = sems[2 * NBUF:3 * NBUF]
        wid = lax.axis_index("s") * nc + lax.axis_index("c")
        wch = wid * n_ch

        def idx_copy(g, slot):
            return pltpu.make_async_copy(
                idx_hbm.at[pl.ds(wch + g, 1)], idx_v.at[pl.ds(slot, 1)],
                isem[slot])

        def gather_copy(g, slot):
            return pltpu.make_async_copy(
                table_hbm.at[idx_v.at[slot]], rows_v.at[slot], gsem[slot])

        def wb_copy(g, slot):
            return pltpu.make_async_copy(
                rows_v.at[slot],
                out_hbm.at[pl.ds((wch + g) * CHUNK, CHUNK)], wsem[slot])

        def visit(g, j, do_idx, do_gather, wait_wb):
            gather_copy(g, j).wait()
            wb_copy(g, j).start()
            if do_idx:
                idx_copy(g + NBUF, j).start()
            if do_gather:
                j2 = (j + KG) % NBUF
                if wait_wb:
                    wb_copy(g + KG - NBUF, j2).wait()
                idx_copy(g + KG, j2).wait()
                gather_copy(g + KG, j2).start()

        for f in range(NBUF):
            idx_copy(f, f).start()
        for f in range(KG):
            idx_copy(f, f).wait()
            gather_copy(f, f).start()
        for g in range(NBUF):
            visit(g, g, True, True, g + KG - NBUF >= 0)

        n_main = (n_ch - 2 * NBUF) // NBUF

        def body(r, carry):
            for j in range(NBUF):
                visit(r * NBUF + j, j, True, True, True)
            return carry

        lax.fori_loop(1, 1 + n_main, body, 0)

        for g in range((1 + n_main) * NBUF, n_ch):
            visit(g, g % NBUF, g + NBUF < n_ch, g + KG < n_ch, True)
        for g in range(n_ch - NBUF, n_ch):
            wb_copy(g, g % NBUF).wait()

    return gather


def kernel(x, table):
    b, h = x.shape
    _, d = table.shape
    n_rows = b * h
    idx = x.reshape(n_rows // CHUNK, CHUNK).astype(jnp.int32)
    out = _build(n_rows, d)(idx, table)
    return out.reshape(b, h, d)
